# baseline (speedup 1.0000x reference)
import jax
import jax.numpy as jnp
from jax.experimental import pallas as pl
from jax.experimental.pallas import tpu as pltpu

_BK = 1024


def _mlp_block(x_ref, w1_ref, b1_ref, w2_ref, b2_ref, out_ref):
    xb = x_ref[...].astype(jnp.bfloat16)
    h = jnp.dot(xb, w1_ref[...], preferred_element_type=jnp.float32)
    h = jnp.maximum(h + b1_ref[...], 0.0).astype(jnp.bfloat16)
    out = jnp.dot(h, w2_ref[...], preferred_element_type=jnp.float32)
    out_ref[...] = out + b2_ref[...]


def kernel(x, W1, b1, W2, b2):
    B, D = x.shape
    H = W1.shape[1]
    b1r = b1.reshape(1, H)
    b2r = b2.reshape(1, D)
    W1b = W1.astype(jnp.bfloat16)
    W2b = W2.astype(jnp.bfloat16)
    grid = (B // _BK,)
    encoded = pl.pallas_call(
        _mlp_block,
        grid=grid,
        in_specs=[
            pl.BlockSpec((_BK, D), lambda i: (i, 0)),
            pl.BlockSpec((D, H), lambda i: (0, 0)),
            pl.BlockSpec((1, H), lambda i: (0, 0)),
            pl.BlockSpec((H, D), lambda i: (0, 0)),
            pl.BlockSpec((1, D), lambda i: (0, 0)),
        ],
        out_specs=pl.BlockSpec((_BK, D), lambda i: (i, 0)),
        out_shape=jax.ShapeDtypeStruct((B, D), x.dtype),
        compiler_params=pltpu.CompilerParams(
            dimension_semantics=("parallel",),
        ),
    )(x, W1b, b1r, W2b, b2r)
    novelty_score = jnp.ones((B, 1), dtype=x.dtype)
    return (novelty_score, encoded)


# speedup vs baseline: 1.5700x; 1.5700x over previous
import jax
import jax.numpy as jnp
from jax.experimental import pallas as pl
from jax.experimental.pallas import tpu as pltpu

_BK = 2048


def _mlp_block(x_ref, w1_ref, b1_ref, w2_ref, b2_ref, out_ref):
    h = jnp.dot(x_ref[...], w1_ref[...], preferred_element_type=jnp.float32)
    h = jnp.maximum(h + b1_ref[...], 0.0)
    out = jnp.dot(h, w2_ref[...], preferred_element_type=jnp.float32)
    out_ref[...] = out + b2_ref[...]


def kernel(x, W1, b1, W2, b2):
    B, D = x.shape
    H = W1.shape[1]
    b1r = b1.reshape(1, H)
    b2r = b2.reshape(1, D)
    grid = (B // _BK,)
    encoded = pl.pallas_call(
        _mlp_block,
        grid=grid,
        in_specs=[
            pl.BlockSpec((_BK, D), lambda i: (i, 0)),
            pl.BlockSpec((D, H), lambda i: (0, 0)),
            pl.BlockSpec((1, H), lambda i: (0, 0)),
            pl.BlockSpec((H, D), lambda i: (0, 0)),
            pl.BlockSpec((1, D), lambda i: (0, 0)),
        ],
        out_specs=pl.BlockSpec((_BK, D), lambda i: (i, 0)),
        out_shape=jax.ShapeDtypeStruct((B, D), x.dtype),
        compiler_params=pltpu.CompilerParams(
            dimension_semantics=("parallel",),
        ),
    )(x, W1, b1r, W2, b2r)
    novelty_score = jnp.ones((B, 1), dtype=x.dtype)
    return (novelty_score, encoded)


# speedup vs baseline: 1.8705x; 1.1913x over previous
import jax
import jax.numpy as jnp
from jax.experimental import pallas as pl
from jax.experimental.pallas import tpu as pltpu

_BK = 4096


def _mlp_block(x_ref, w1_ref, b1_ref, w2_ref, b2_ref, out_ref):
    h = jnp.dot(x_ref[...], w1_ref[...], preferred_element_type=jnp.float32)
    h = jnp.maximum(h + b1_ref[...], 0.0)
    out = jnp.dot(h, w2_ref[...], preferred_element_type=jnp.float32)
    out_ref[...] = out + b2_ref[...]


def kernel(x, W1, b1, W2, b2):
    B, D = x.shape
    H = W1.shape[1]
    b1r = b1.reshape(1, H)
    b2r = b2.reshape(1, D)
    grid = (B // _BK,)
    encoded = pl.pallas_call(
        _mlp_block,
        grid=grid,
        in_specs=[
            pl.BlockSpec((_BK, D), lambda i: (i, 0)),
            pl.BlockSpec((D, H), lambda i: (0, 0)),
            pl.BlockSpec((1, H), lambda i: (0, 0)),
            pl.BlockSpec((H, D), lambda i: (0, 0)),
            pl.BlockSpec((1, D), lambda i: (0, 0)),
        ],
        out_specs=pl.BlockSpec((_BK, D), lambda i: (i, 0)),
        out_shape=jax.ShapeDtypeStruct((B, D), x.dtype),
        compiler_params=pltpu.CompilerParams(
            dimension_semantics=("parallel",),
        ),
    )(x, W1, b1r, W2, b2r)
    novelty_score = jnp.ones((B, 1), dtype=x.dtype)
    return (novelty_score, encoded)


# speedup vs baseline: 1.9445x; 1.0396x over previous
import jax
import jax.numpy as jnp
from jax.experimental import pallas as pl
from jax.experimental.pallas import tpu as pltpu

_BK = 8192


def _mlp_block(x_ref, w1_ref, b1_ref, w2_ref, b2_ref, out_ref):
    h = jnp.dot(x_ref[...], w1_ref[...], preferred_element_type=jnp.float32)
    h = jnp.maximum(h + b1_ref[...], 0.0)
    out = jnp.dot(h, w2_ref[...], preferred_element_type=jnp.float32)
    out_ref[...] = out + b2_ref[...]


def kernel(x, W1, b1, W2, b2):
    B, D = x.shape
    H = W1.shape[1]
    b1r = b1.reshape(1, H)
    b2r = b2.reshape(1, D)
    grid = (B // _BK,)
    encoded = pl.pallas_call(
        _mlp_block,
        grid=grid,
        in_specs=[
            pl.BlockSpec((_BK, D), lambda i: (i, 0)),
            pl.BlockSpec((D, H), lambda i: (0, 0)),
            pl.BlockSpec((1, H), lambda i: (0, 0)),
            pl.BlockSpec((H, D), lambda i: (0, 0)),
            pl.BlockSpec((1, D), lambda i: (0, 0)),
        ],
        out_specs=pl.BlockSpec((_BK, D), lambda i: (i, 0)),
        out_shape=jax.ShapeDtypeStruct((B, D), x.dtype),
        compiler_params=pltpu.CompilerParams(
            dimension_semantics=("parallel",),
        ),
    )(x, W1, b1r, W2, b2r)
    novelty_score = jnp.ones((B, 1), dtype=x.dtype)
    return (novelty_score, encoded)
